# baseline (speedup 1.0000x reference)
import functools

import jax
import jax.numpy as jnp
from jax import lax
from jax.experimental import pallas as pl
from jax.experimental.pallas import tpu as pltpu
from jax.experimental.pallas import tpu_sc as plsc

B = 16384
T = 1284
W = 2576
G = 8
R = 161
NR = 16 * W // G
NC = 2
NS = 16
NW = NC * NS
BPW = B // NW
C = 16
H = BPW // C
CI = C * R


def _annual_table_tc(qx, kx):
    def body(qx_ref, kx_ref, o_ref):
        o_ref[...] = (jnp.power(qx_ref[...] + 1.0, 1.0 / 12.0) - 1.0) * (
            1.0 - kx_ref[...]
        )

    return pl.pallas_call(
        body,
        out_shape=jax.ShapeDtypeStruct(qx.shape, jnp.float32),
    )(qx, kx)


_IDX_BLK = 2048


def _window_indices_tc(sex, age, dur):
    def body(sex_ref, age_ref, dur_ref, o_ref):
        mth = age_ref[...] * 12 + dur_ref[...]
        p = jnp.bitwise_and(mth, G - 1)
        start8 = (p * 2 + sex_ref[...]) * (W // G) + lax.shift_right_logical(
            mth - p, 3
        )
        o_ref[...] = start8 + lax.broadcasted_iota(
            jnp.int32, (_IDX_BLK, R), 1
        )

    spec1 = pl.BlockSpec((_IDX_BLK, 1), lambda i: (i, 0))
    return pl.pallas_call(
        body,
        grid=(B // _IDX_BLK,),
        in_specs=[spec1, spec1, spec1],
        out_specs=pl.BlockSpec((_IDX_BLK, R), lambda i: (i, 0)),
        out_shape=jax.ShapeDtypeStruct((B, R), jnp.int32),
    )(sex.reshape(B, 1), age.reshape(B, 1), dur.reshape(B, 1))


def _make_sc_kernel(interpret=False):
    mesh = plsc.VectorSubcoreMesh(core_axis_name="c", subcore_axis_name="s")

    @functools.partial(
        pl.kernel,
        out_type=jax.ShapeDtypeStruct((B * R, G), jnp.float32),
        mesh=mesh,
        interpret=interpret,
        compiler_params=pltpu.CompilerParams(use_tc_tiling_on_sc=False),
        scratch_types=[
            pltpu.VMEM_SHARED((NR, G), jnp.float32),
            pltpu.VMEM((2, CI), jnp.int32),
            pltpu.VMEM((2, CI, G), jnp.float32),
        ]
        + [pltpu.SemaphoreType.DMA] * 6,
    )
    def sc_kern(t8_hbm, idx_hbm, out_hbm, table_sh, idx_v, chunk_v, *sems):
        cidx = lax.axis_index("c")
        sid = lax.axis_index("s")
        wid = sid * NC + cidx
        base = wid * BPW

        @pl.when(sid == 0)
        def _():
            pltpu.sync_copy(t8_hbm, table_sh)

        plsc.subcore_barrier()

        isem = sems[0:2]
        gsem = sems[2:4]
        osem = sems[4:6]

        def istart(d, h):
            pltpu.make_async_copy(
                idx_hbm.at[pl.ds((base + h * C) * R, CI)],
                idx_v.at[d],
                isem[d],
            ).start()

        def iwait(d):
            pltpu.make_async_copy(
                idx_hbm.at[pl.ds(0, CI)],
                idx_v.at[d],
                isem[d],
            ).wait()

        def gstart(d):
            pltpu.make_async_copy(
                table_sh.at[idx_v.at[d]],
                chunk_v.at[d],
                gsem[d],
            ).start()

        def gwait(d):
            pltpu.make_async_copy(
                table_sh.at[idx_v.at[d]],
                chunk_v.at[d],
                gsem[d],
            ).wait()

        def ostart(d, h):
            pltpu.make_async_copy(
                chunk_v.at[d],
                out_hbm.at[pl.ds((base + h * C) * R, CI)],
                osem[d],
            ).start()

        def owait(d):
            pltpu.make_async_copy(
                chunk_v.at[d],
                out_hbm.at[pl.ds(0, CI)],
                osem[d],
            ).wait()

        def body(h, carry):
            istart(0, h)
            iwait(0)
            gstart(0)
            gwait(0)
            ostart(0, h)
            owait(0)
            return carry

        lax.fori_loop(0, H, body, 0)

    return sc_kern


_SC_KERN = _make_sc_kernel()


def kernel(mp_idx, mp_val, qx, kx):
    q_ann = _annual_table_tc(qx, kx)
    q_mth = jnp.repeat(q_ann, 12, axis=1)
    t_pad = jnp.zeros((2, W + G), jnp.float32).at[:, :T].set(q_mth)
    t8 = jnp.stack([t_pad[:, p : p + W] for p in range(G)])
    t8_rows = t8.reshape(NR, G)
    idx = _window_indices_tc(mp_idx[:, 0], mp_idx[:, 1], mp_idx[:, 4])
    out = _SC_KERN(t8_rows, idx.reshape(B * R))
    return out.reshape(B, R * G)[:, :T]


# speedup vs baseline: 1.1624x; 1.1624x over previous
import functools

import jax
import jax.numpy as jnp
from jax import lax
from jax.experimental import pallas as pl
from jax.experimental.pallas import tpu as pltpu
from jax.experimental.pallas import tpu_sc as plsc

B = 16384
T = 1284
W = 2576
G = 8
R = 161
NR = 16 * W // G
NC = 2
NS = 16
NW = NC * NS
BPW = B // NW
C = 16
H = BPW // C
CI = C * R


def _annual_table_tc(qx, kx):
    def body(qx_ref, kx_ref, o_ref):
        o_ref[...] = (jnp.power(qx_ref[...] + 1.0, 1.0 / 12.0) - 1.0) * (
            1.0 - kx_ref[...]
        )

    return pl.pallas_call(
        body,
        out_shape=jax.ShapeDtypeStruct(qx.shape, jnp.float32),
    )(qx, kx)


_IDX_BLK = 2048


def _window_indices_tc(sex, age, dur):
    def body(sex_ref, age_ref, dur_ref, o_ref):
        mth = age_ref[...] * 12 + dur_ref[...]
        p = jnp.bitwise_and(mth, G - 1)
        start8 = (p * 2 + sex_ref[...]) * (W // G) + lax.shift_right_logical(
            mth - p, 3
        )
        o_ref[...] = start8 + lax.broadcasted_iota(
            jnp.int32, (_IDX_BLK, R), 1
        )

    spec1 = pl.BlockSpec((_IDX_BLK, 1), lambda i: (i, 0))
    return pl.pallas_call(
        body,
        grid=(B // _IDX_BLK,),
        in_specs=[spec1, spec1, spec1],
        out_specs=pl.BlockSpec((_IDX_BLK, R), lambda i: (i, 0)),
        out_shape=jax.ShapeDtypeStruct((B, R), jnp.int32),
    )(sex.reshape(B, 1), age.reshape(B, 1), dur.reshape(B, 1))


def _make_sc_kernel(interpret=False):
    mesh = plsc.VectorSubcoreMesh(core_axis_name="c", subcore_axis_name="s")

    @functools.partial(
        pl.kernel,
        out_type=jax.ShapeDtypeStruct((B * R, G), jnp.float32),
        mesh=mesh,
        interpret=interpret,
        compiler_params=pltpu.CompilerParams(use_tc_tiling_on_sc=False),
        scratch_types=[
            pltpu.VMEM_SHARED((NR, G), jnp.float32),
            pltpu.VMEM((2, CI), jnp.int32),
            pltpu.VMEM((2, CI, G), jnp.float32),
        ]
        + [pltpu.SemaphoreType.DMA] * 6,
    )
    def sc_kern(t8_hbm, idx_hbm, out_hbm, table_sh, idx_v, chunk_v, *sems):
        cidx = lax.axis_index("c")
        sid = lax.axis_index("s")
        wid = sid * NC + cidx
        base = wid * BPW

        @pl.when(sid == 0)
        def _():
            pltpu.sync_copy(t8_hbm, table_sh)

        plsc.subcore_barrier()

        isem = sems[0:2]
        gsem = sems[2:4]
        osem = sems[4:6]

        def istart(d, h):
            pltpu.make_async_copy(
                idx_hbm.at[pl.ds((base + h * C) * R, CI)],
                idx_v.at[d],
                isem[d],
            ).start()

        def iwait(d):
            pltpu.make_async_copy(
                idx_hbm.at[pl.ds(0, CI)],
                idx_v.at[d],
                isem[d],
            ).wait()

        def gstart(d):
            pltpu.make_async_copy(
                table_sh.at[idx_v.at[d]],
                chunk_v.at[d],
                gsem[d],
            ).start()

        def gwait(d):
            pltpu.make_async_copy(
                table_sh.at[idx_v.at[d]],
                chunk_v.at[d],
                gsem[d],
            ).wait()

        def ostart(d, h):
            pltpu.make_async_copy(
                chunk_v.at[d],
                out_hbm.at[pl.ds((base + h * C) * R, CI)],
                osem[d],
            ).start()

        def owait(d):
            pltpu.make_async_copy(
                chunk_v.at[d],
                out_hbm.at[pl.ds(0, CI)],
                osem[d],
            ).wait()

        istart(0, 0)
        istart(1, 1)

        def step(d, h, hh):
            iwait(d)

            @pl.when(hh >= 1)
            def _():
                owait(d)

            gstart(d)
            gwait(d)
            ostart(d, h)

            @pl.when(hh <= H // 2 - 2)
            def _():
                istart(d, h + 2)

        def body(hh, carry):
            step(0, hh * 2, hh)
            step(1, hh * 2 + 1, hh)
            return carry

        lax.fori_loop(0, H // 2, body, 0)
        owait(0)
        owait(1)

    return sc_kern


_SC_KERN = _make_sc_kernel()


def kernel(mp_idx, mp_val, qx, kx):
    q_ann = _annual_table_tc(qx, kx)
    q_mth = jnp.repeat(q_ann, 12, axis=1)
    t_pad = jnp.zeros((2, W + G), jnp.float32).at[:, :T].set(q_mth)
    t8 = jnp.stack([t_pad[:, p : p + W] for p in range(G)])
    t8_rows = t8.reshape(NR, G)
    idx = _window_indices_tc(mp_idx[:, 0], mp_idx[:, 1], mp_idx[:, 4])
    out = _SC_KERN(t8_rows, idx.reshape(B * R))
    return out.reshape(B, R * G)[:, :T]
